# Initial kernel scaffold; baseline (speedup 1.0000x reference)
import jax
import jax.numpy as jnp
from jax.experimental import pallas as pl


def kernel(h, x, w, Wx_v, bx_v, Ww_v, bw_v, Wx_e, bx_e, Ww_e, bw_e):
    raise NotImplementedError("write your pallas kernel here")


# speedup vs baseline: 7.1290x; 7.1290x over previous
import functools

import jax
import jax.numpy as jnp
from jax import lax
from jax.experimental import pallas as pl
from jax.experimental.pallas import tpu as pltpu
from jax.experimental.pallas import tpu_sc as plsc

_N = 10000
_E = 320000
_D = 128

_NC = 2
_NS = 16
_NW = _NC * _NS
_EPW = _E // _NW
_K = 80
_CH = _EPW // _K
_NP = 10240
_RPT = _NP // _NS
_RCHUNK = 128


_ROWS = 2000


def _lin3_body(x_ref, w_ref, Wxv_ref, bxv_ref, Wwv_ref, bwv_ref,
               Wwe_ref, bwe_ref, x1_ref, w1_ref, w2_ref):
    x = x_ref[...]
    w = w_ref[...]
    x1_ref[...] = jnp.dot(x, Wxv_ref[...],
                          preferred_element_type=jnp.float32) + bxv_ref[...]
    w1_ref[...] = jnp.dot(w, Wwv_ref[...],
                          preferred_element_type=jnp.float32) + bwv_ref[...]
    w2_ref[...] = jnp.dot(w, Wwe_ref[...],
                          preferred_element_type=jnp.float32) + bwe_ref[...]


def _update_lin_body(x1_ref, p0_ref, p1_ref, Wxe_ref, bxe_ref,
                     xn_ref, x2_ref):
    x1 = x1_ref[...]
    xn = x1 + (p0_ref[...] + p1_ref[...]) * x1
    xn_ref[...] = xn
    x2_ref[...] = jnp.dot(xn, Wxe_ref[...],
                          preferred_element_type=jnp.float32) + bxe_ref[...]


def _update_body(w2_ref, q0_ref, q1_ref, wn_ref):
    w2 = w2_ref[...]
    wn_ref[...] = w2 + (q0_ref[...] + q1_ref[...]) * w2


def _row_spec():
    return pl.BlockSpec((_ROWS, _D), lambda i: (i, 0))


def _full_spec(shape):
    return pl.BlockSpec(shape, lambda i: tuple(0 for _ in shape))


def _sc_aggregate(table, gidx, sidx):
    mesh = plsc.VectorSubcoreMesh(core_axis_name="c", subcore_axis_name="s")

    @functools.partial(
        pl.kernel,
        mesh=mesh,
        out_type=jax.ShapeDtypeStruct((_NC * _NP, _D), jnp.float32),
        scratch_types=[
            pltpu.VMEM((_CH, _K), jnp.int32),
            pltpu.VMEM((_CH, _K), jnp.int32),
            pltpu.VMEM((_RCHUNK, _D), jnp.float32),
            pltpu.VMEM_SHARED((_NP, _D), jnp.float32),
            pltpu.SemaphoreType.DMA,
        ],
    )
    def body(table_hbm, gidx_hbm, sidx_hbm, out_hbm,
             gidx_v, sidx_v, buf_v, accum_sh, sem):
        c = lax.axis_index("c")
        s = lax.axis_index("s")
        wid = s * _NC + c

        def zrow(i, _):
            for j in range(_D // 16):
                buf_v[i, pl.ds(j * 16, 16)] = jnp.zeros((16,), jnp.float32)
            return _
        lax.fori_loop(0, _RCHUNK, zrow, None)
        for r in range(_RPT // _RCHUNK):
            pltpu.sync_copy(
                buf_v, accum_sh.at[pl.ds(s * _RPT + r * _RCHUNK, _RCHUNK)])
        plsc.subcore_barrier()

        pltpu.sync_copy(gidx_hbm.at[wid], gidx_v)
        pltpu.sync_copy(sidx_hbm.at[wid], sidx_v)

        def chunk(j, _):
            rows = buf_v.at[pl.ds(0, _K)]
            pltpu.async_copy(table_hbm.at[gidx_v.at[j]], rows, sem).wait()
            pltpu.sync_copy(rows, accum_sh.at[sidx_v.at[j]], add=True)
            return _
        lax.fori_loop(0, _CH, chunk, None)
        plsc.subcore_barrier()

        pltpu.sync_copy(accum_sh.at[pl.ds(s * _RPT, _RPT)],
                        out_hbm.at[pl.ds(c * _NP + s * _RPT, _RPT)])

    return body(table, gidx, sidx)


def kernel(h, x, w, Wx_v, bx_v, Ww_v, bw_v, Wx_e, bx_e, Ww_e, bw_e):
    h32 = h.astype(jnp.int32)
    src = h32[0].reshape(_NW, _CH, _K)
    dst = h32[1].reshape(_NW, _CH, _K)
    bx_v2 = bx_v.reshape(1, _D)
    bw_v2 = bw_v.reshape(1, _D)
    bx_e2 = bx_e.reshape(1, _D)
    bw_e2 = bw_e.reshape(1, _D)

    wspec = _full_spec((_D, _D))
    bspec = _full_spec((1, _D))

    x1, w1, w2 = pl.pallas_call(
        _lin3_body,
        grid=(_N // _ROWS,),
        in_specs=[_row_spec(), _row_spec(),
                  wspec, bspec, wspec, bspec, wspec, bspec],
        out_specs=[_row_spec(), _row_spec(), _row_spec()],
        out_shape=[jax.ShapeDtypeStruct((_N, _D), jnp.float32)] * 3,
    )(x, w, Wx_v, bx_v2, Ww_v, bw_v2, Ww_e, bw_e2)

    pv = _sc_aggregate(w1, dst, src)

    x_new, x2 = pl.pallas_call(
        _update_lin_body,
        grid=(_N // _ROWS,),
        in_specs=[_row_spec(), _row_spec(), _row_spec(), wspec, bspec],
        out_specs=[_row_spec(), _row_spec()],
        out_shape=[jax.ShapeDtypeStruct((_N, _D), jnp.float32)] * 2,
    )(x1, pv[:_N], pv[_NP:_NP + _N], Wx_e, bx_e2)

    qv = _sc_aggregate(x2, src, dst)

    w_new = pl.pallas_call(
        _update_body,
        grid=(_N // _ROWS,),
        in_specs=[_row_spec(), _row_spec(), _row_spec()],
        out_specs=_row_spec(),
        out_shape=jax.ShapeDtypeStruct((_N, _D), jnp.float32),
    )(w2, qv[:_N], qv[_NP:_NP + _N])

    return (w_new, x_new)
